# baseline (speedup 1.0000x reference)
import functools

import jax
import jax.numpy as jnp
from jax.experimental import pallas as pl

N = 50000
M = 6250
K3 = 27
K2 = 8


def _mm_kernel(x_ref, w_ref, o_ref):
    o_ref[...] = jax.lax.dot_general(
        x_ref[...], w_ref[...], (((1,), (0,)), ((), ())),
        preferred_element_type=jnp.float32)


def _einsum(x, w2d, bn):
    R, C = x.shape
    KO = w2d.shape[1]
    if bn is None or R % bn != 0:
        bn = R
    grid = R // bn
    return pl.pallas_call(
        _mm_kernel,
        grid=(grid,),
        in_specs=[pl.BlockSpec((bn, C), lambda i: (i, 0)),
                  pl.BlockSpec((C, KO), lambda i: (0, 0))],
        out_specs=pl.BlockSpec((bn, KO), lambda i: (i, 0)),
        out_shape=jax.ShapeDtypeStruct((R, KO), jnp.float32),
    )(x, w2d)


def _w2d(W):
    K, C, O = W.shape
    return W.transpose(1, 0, 2).reshape(C, K * O)


def _sparse_conv(x, W, b, src, dst, kid, n_out, bn):
    K, C, O = W.shape
    h = _einsum(x, _w2d(W), bn).reshape(-1, O)
    msg = h[src * K + kid]
    out = jax.ops.segment_sum(msg, dst, num_segments=n_out)
    return out + b


def kernel(x, edge_index1, kid1, src_d, dst_d, kid_d, edge_index3, kid3,
           W1, b1, Wd, bd, Wra, bra, Wrb, brb, W4, b4):
    BL = Wra.shape[0]
    out = _sparse_conv(x, W1, b1, edge_index1[0], edge_index1[1], kid1, N, 2000)
    out = jax.nn.relu(out)
    out = _sparse_conv(out, Wd, bd, src_d, dst_d, kid_d, M, 2000)
    out = jax.nn.relu(out)
    src3, dst3 = edge_index3[0], edge_index3[1]
    for i in range(BL):
        h = _sparse_conv(out, Wra[i], bra[i], src3, dst3, kid3, M, None)
        h = jax.nn.relu(h)
        h = _sparse_conv(h, Wrb[i], brb[i], src3, dst3, kid3, M, None)
        out = jax.nn.relu(out + h)
    out = _sparse_conv(out, W4, b4, src3, dst3, kid3, M, None)
    return out


# speedup vs baseline: 2.2250x; 2.2250x over previous
import functools

import jax
import jax.numpy as jnp
from jax import lax
from jax.experimental import pallas as pl
from jax.experimental.pallas import tpu as pltpu, tpu_sc as plsc

N = 50000
M = 6250
K3 = 27
K2 = 8
NC = 2
NS = 16
CHUNK = 128

_F32 = jnp.float32
_DN = (((1,), (0,)), ((), ()))


def _ka_kernel(x_ref, w_ref, o_ref):
    o_ref[0] = lax.dot_general(x_ref[...], w_ref[0], _DN,
                               preferred_element_type=_F32)


def _conv1_tables(x, w_stacked, bn):
    nb = N // bn
    return pl.pallas_call(
        _ka_kernel,
        grid=(NC, nb),
        in_specs=[pl.BlockSpec((bn, 64), lambda c, i: (i, 0)),
                  pl.BlockSpec((1, 64, 27 * 32), lambda c, i: (c, 0, 0))],
        out_specs=pl.BlockSpec((1, bn, 27 * 32), lambda c, i: (c, i, 0)),
        out_shape=jax.ShapeDtypeStruct((NC, N, 27 * 32), _F32),
    )(x, w_stacked)


def _kb_down_kernel(pa_ref, pb_ref, ba_ref, bb_ref, wa_ref, wb_ref, o_ref):
    ta = jnp.maximum(pa_ref[...] + ba_ref[...], 0.0)
    tb = jnp.maximum(pb_ref[...] + bb_ref[...], 0.0)
    o_ref[...] = (
        lax.dot_general(ta, wa_ref[...], _DN, preferred_element_type=_F32)
        + lax.dot_general(tb, wb_ref[...], _DN, preferred_element_type=_F32))


def _kb_down(pa, pb, ba, bb, wa, wb, bn):
    nb = N // bn
    ko = wa.shape[1]
    return pl.pallas_call(
        _kb_down_kernel,
        grid=(nb,),
        in_specs=[pl.BlockSpec((bn, 32), lambda i: (i, 0)),
                  pl.BlockSpec((bn, 32), lambda i: (i, 0)),
                  pl.BlockSpec((1, 32), lambda i: (0, 0)),
                  pl.BlockSpec((1, 32), lambda i: (0, 0)),
                  pl.BlockSpec((32, ko), lambda i: (0, 0)),
                  pl.BlockSpec((32, ko), lambda i: (0, 0))],
        out_specs=pl.BlockSpec((bn, ko), lambda i: (i, 0)),
        out_shape=jax.ShapeDtypeStruct((N, ko), _F32),
    )(pa, pb, ba, bb, wa, wb)


def _kb_first_kernel(q0_ref, q1_ref, b_ref, w_ref, h_ref, cur_ref):
    cur = jnp.maximum(q0_ref[...] + q1_ref[...] + b_ref[...], 0.0)
    cur_ref[...] = cur
    h_ref[...] = lax.dot_general(cur, w_ref[...], _DN,
                                 preferred_element_type=_F32)


def _kb_first(q0, q1, b, w):
    ko = w.shape[1]
    return pl.pallas_call(
        _kb_first_kernel,
        out_shape=(jax.ShapeDtypeStruct((M, ko), _F32),
                   jax.ShapeDtypeStruct((M, 32), _F32)),
    )(q0, q1, b, w)


def _kb_mid_kernel(r0_ref, r1_ref, b_ref, w_ref, h_ref):
    t = jnp.maximum(r0_ref[...] + r1_ref[...] + b_ref[...], 0.0)
    h_ref[...] = lax.dot_general(t, w_ref[...], _DN,
                                 preferred_element_type=_F32)


def _kb_mid(r0, r1, b, w):
    ko = w.shape[1]
    return pl.pallas_call(
        _kb_mid_kernel,
        out_shape=jax.ShapeDtypeStruct((M, ko), _F32),
    )(r0, r1, b, w)


def _kb_res_kernel(cur_ref, r0_ref, r1_ref, b_ref, w_ref, h_ref, cur2_ref):
    cur2 = jnp.maximum(cur_ref[...] + r0_ref[...] + r1_ref[...] + b_ref[...],
                       0.0)
    cur2_ref[...] = cur2
    h_ref[...] = lax.dot_general(cur2, w_ref[...], _DN,
                                 preferred_element_type=_F32)


def _kb_res(cur, r0, r1, b, w):
    ko = w.shape[1]
    return pl.pallas_call(
        _kb_res_kernel,
        out_shape=(jax.ShapeDtypeStruct((M, ko), _F32),
                   jax.ShapeDtypeStruct((M, 32), _F32)),
    )(cur, r0, r1, b, w)


def _kf_kernel(e0_ref, e1_ref, b_ref, o_ref):
    o_ref[...] = e0_ref[...] + e1_ref[...] + b_ref[...]


def _kf(e0, e1, b):
    return pl.pallas_call(
        _kf_kernel,
        out_shape=jax.ShapeDtypeStruct((M, 16), _F32),
    )(e0, e1, b)


def _mesh():
    return plsc.VectorSubcoreMesh(core_axis_name="c", subcore_axis_name="s",
                                  num_cores=NC, num_subcores=NS)


def _sc_edge_conv(table, src, kid, dst, zeros, *, K, acc_rows, C,
                  split_cores, idx_core_stride):
    Ep = src.shape[0]
    n_workers = NC * NS if split_cores else NS
    per_w = Ep // n_workers
    nch = per_w // CHUNK
    assert per_w % CHUNK == 0 and acc_rows % NS == 0
    rps = acc_rows // NS

    @functools.partial(
        pl.kernel,
        out_type=jax.ShapeDtypeStruct((NC * acc_rows, C), _F32),
        mesh=_mesh(),
        scratch_types=[
            pltpu.VMEM((CHUNK,), jnp.int32),
            pltpu.VMEM((CHUNK,), jnp.int32),
            pltpu.VMEM((CHUNK,), jnp.int32),
            pltpu.VMEM((CHUNK,), jnp.int32),
            pltpu.VMEM((CHUNK, C), _F32),
            pltpu.VMEM_SHARED((acc_rows, C), _F32),
            pltpu.SemaphoreType.DMA,
        ],
        compiler_params=pltpu.CompilerParams(use_tc_tiling_on_sc=False),
    )
    def k(table_h, src_h, kid_h, dst_h, zeros_h, out_h,
          src_v, kid_v, dst_v, idx_v, rows_v, acc, sem):
        c = lax.axis_index("c")
        s = lax.axis_index("s")
        pltpu.sync_copy(zeros_h.at[pl.ds(0, rps)],
                        acc.at[pl.ds(s * rps, rps)])
        plsc.subcore_barrier()
        if split_cores:
            base = (s * NC + c) * per_w
        else:
            base = s * per_w

        def body(t, carry):
            e0 = base + t * CHUNK
            pltpu.sync_copy(src_h.at[pl.ds(e0, CHUNK)], src_v)
            pltpu.sync_copy(kid_h.at[pl.ds(e0, CHUNK)], kid_v)
            pltpu.sync_copy(dst_h.at[pl.ds(e0, CHUNK)], dst_v)
            for j in range(CHUNK // 16):
                sl = pl.ds(j * 16, 16)
                idx = src_v[sl] * K + kid_v[sl]
                if idx_core_stride:
                    idx = idx + c * idx_core_stride
                idx_v[sl] = idx
            pltpu.async_copy(table_h.at[idx_v], rows_v, sem).wait()
            pltpu.sync_copy(rows_v, acc.at[dst_v], add=True)
            return carry

        lax.fori_loop(0, nch, body, 0)
        plsc.subcore_barrier()
        pltpu.sync_copy(acc.at[pl.ds(s * rps, rps)],
                        out_h.at[pl.ds(c * acc_rows + s * rps, rps)])

    return k(table, src, kid, dst, zeros)


def _pad_edges(src, kid, dst, n_workers, dummy_row):
    per = n_workers * CHUNK
    E = src.shape[0]
    Ep = ((E + per - 1) // per) * per
    pad = Ep - E
    src = jnp.concatenate([src, jnp.zeros((pad,), jnp.int32)])
    kid = jnp.concatenate([kid, jnp.zeros((pad,), jnp.int32)])
    dst = jnp.concatenate([dst, jnp.full((pad,), dummy_row, jnp.int32)])
    return src, kid, dst


def kernel(x, edge_index1, kid1, src_d, dst_d, kid_d, edge_index3, kid3,
           W1, b1, Wd, bd, Wra, bra, Wrb, brb, W4, b4):
    BL = Wra.shape[0]
    NPAD = 50048
    MPAD = 6272

    w1_stacked = jnp.stack([
        W1[:, :, :32].transpose(1, 0, 2).reshape(64, 27 * 32),
        W1[:, :, 32:].transpose(1, 0, 2).reshape(64, 27 * 32)])
    wd2d = Wd.transpose(1, 0, 2).reshape(64, K2 * 32)
    wra2d = Wra.transpose(0, 2, 1, 3).reshape(BL, 32, 27 * 32)
    wrb2d = Wrb.transpose(0, 2, 1, 3).reshape(BL, 32, 27 * 32)
    w4p = jnp.pad(W4.transpose(1, 0, 2), ((0, 0), (0, 0), (0, 8)))
    w4p = w4p.reshape(32, 27 * 16)
    b1a = b1[:32].reshape(1, 32)
    b1b = b1[32:].reshape(1, 32)
    bd2 = bd.reshape(1, 32)
    b4p = jnp.pad(b4, (0, 8)).reshape(1, 16)
    zeros32 = jnp.zeros((NPAD // NS, 32), _F32)
    zeros16 = jnp.zeros((MPAD // NS, 16), _F32)

    s1, k1, d1 = _pad_edges(edge_index1[0], kid1, edge_index1[1], NS, N)
    sd, kd, dd = _pad_edges(src_d, kid_d, dst_d, NC * NS, M)
    s3, k3, d3 = _pad_edges(edge_index3[0], kid3, edge_index3[1], NC * NS, M)

    h1 = _conv1_tables(x, w1_stacked, 2000)
    table1 = h1.reshape(NC * N * K3, 32)
    p1 = _sc_edge_conv(table1, s1, k1, d1, zeros32, K=K3, acc_rows=NPAD,
                       C=32, split_cores=False, idx_core_stride=N * K3)
    p1 = p1.reshape(NC, NPAD, 32)[:, :N]

    hd = _kb_down(p1[0], p1[1], b1a, b1b, wd2d[:32], wd2d[32:], 2000)
    qd = _sc_edge_conv(hd.reshape(N * K2, 32), sd, kd, dd, zeros32[:MPAD // NS],
                       K=K2, acc_rows=MPAD, C=32, split_cores=True,
                       idx_core_stride=0)
    qd = qd.reshape(NC, MPAD, 32)[:, :M]

    ha, cur = _kb_first(qd[0], qd[1], bd2, wra2d[0])
    for i in range(BL):
        ra = _sc_edge_conv(ha.reshape(M * K3, 32), s3, k3, d3,
                           zeros32[:MPAD // NS], K=K3, acc_rows=MPAD, C=32,
                           split_cores=True, idx_core_stride=0)
        ra = ra.reshape(NC, MPAD, 32)[:, :M]
        hb = _kb_mid(ra[0], ra[1], bra[i].reshape(1, 32), wrb2d[i])
        rb = _sc_edge_conv(hb.reshape(M * K3, 32), s3, k3, d3,
                           zeros32[:MPAD // NS], K=K3, acc_rows=MPAD, C=32,
                           split_cores=True, idx_core_stride=0)
        rb = rb.reshape(NC, MPAD, 32)[:, :M]
        wnext = wra2d[i + 1] if i + 1 < BL else w4p
        ha, cur = _kb_res(cur, rb[0], rb[1], brb[i].reshape(1, 32), wnext)

    e = _sc_edge_conv(ha.reshape(M * K3, 16), s3, k3, d3, zeros16,
                      K=K3, acc_rows=MPAD, C=16, split_cores=True,
                      idx_core_stride=0)
    e = e.reshape(NC, MPAD, 16)[:, :M]
    out = _kf(e[0], e[1], b4p)
    return out[:, :8]


# speedup vs baseline: 3.2926x; 1.4798x over previous
import functools

import jax
import jax.numpy as jnp
from jax import lax
from jax.experimental import pallas as pl
from jax.experimental.pallas import tpu as pltpu, tpu_sc as plsc

N = 50000
M = 6250
K3 = 27
K2 = 8
NC = 2
NS = 16
CHUNK = 128

_F32 = jnp.float32
_DN = (((1,), (0,)), ((), ()))


def _ka_kernel(x_ref, w_ref, o_ref):
    o_ref[0] = lax.dot_general(x_ref[...], w_ref[0], _DN,
                               preferred_element_type=_F32)


def _conv1_tables(x, w_stacked, bn):
    nb = N // bn
    return pl.pallas_call(
        _ka_kernel,
        grid=(NC, nb),
        in_specs=[pl.BlockSpec((bn, 64), lambda c, i: (i, 0)),
                  pl.BlockSpec((1, 64, 27 * 32), lambda c, i: (c, 0, 0))],
        out_specs=pl.BlockSpec((1, bn, 27 * 32), lambda c, i: (c, i, 0)),
        out_shape=jax.ShapeDtypeStruct((NC, N, 27 * 32), _F32),
    )(x, w_stacked)


def _kb_down_kernel(pa_ref, pb_ref, ba_ref, bb_ref, wa_ref, wb_ref, o_ref):
    ta = jnp.maximum(pa_ref[...] + ba_ref[...], 0.0)
    tb = jnp.maximum(pb_ref[...] + bb_ref[...], 0.0)
    o_ref[...] = (
        lax.dot_general(ta, wa_ref[...], _DN, preferred_element_type=_F32)
        + lax.dot_general(tb, wb_ref[...], _DN, preferred_element_type=_F32))


def _kb_down(pa, pb, ba, bb, wa, wb, bn):
    nb = N // bn
    ko = wa.shape[1]
    return pl.pallas_call(
        _kb_down_kernel,
        grid=(nb,),
        in_specs=[pl.BlockSpec((bn, 32), lambda i: (i, 0)),
                  pl.BlockSpec((bn, 32), lambda i: (i, 0)),
                  pl.BlockSpec((1, 32), lambda i: (0, 0)),
                  pl.BlockSpec((1, 32), lambda i: (0, 0)),
                  pl.BlockSpec((32, ko), lambda i: (0, 0)),
                  pl.BlockSpec((32, ko), lambda i: (0, 0))],
        out_specs=pl.BlockSpec((bn, ko), lambda i: (i, 0)),
        out_shape=jax.ShapeDtypeStruct((N, ko), _F32),
    )(pa, pb, ba, bb, wa, wb)


def _kb_first_kernel(q0_ref, q1_ref, b_ref, w_ref, h_ref, cur_ref):
    cur = jnp.maximum(q0_ref[...] + q1_ref[...] + b_ref[...], 0.0)
    cur_ref[...] = cur
    h_ref[...] = lax.dot_general(cur, w_ref[...], _DN,
                                 preferred_element_type=_F32)


def _kb_first(q0, q1, b, w):
    ko = w.shape[1]
    return pl.pallas_call(
        _kb_first_kernel,
        out_shape=(jax.ShapeDtypeStruct((M, ko), _F32),
                   jax.ShapeDtypeStruct((M, 32), _F32)),
    )(q0, q1, b, w)


def _kb_mid_kernel(r0_ref, r1_ref, b_ref, w_ref, h_ref):
    t = jnp.maximum(r0_ref[...] + r1_ref[...] + b_ref[...], 0.0)
    h_ref[...] = lax.dot_general(t, w_ref[...], _DN,
                                 preferred_element_type=_F32)


def _kb_mid(r0, r1, b, w):
    ko = w.shape[1]
    return pl.pallas_call(
        _kb_mid_kernel,
        out_shape=jax.ShapeDtypeStruct((M, ko), _F32),
    )(r0, r1, b, w)


def _kb_res_kernel(cur_ref, r0_ref, r1_ref, b_ref, w_ref, h_ref, cur2_ref):
    cur2 = jnp.maximum(cur_ref[...] + r0_ref[...] + r1_ref[...] + b_ref[...],
                       0.0)
    cur2_ref[...] = cur2
    h_ref[...] = lax.dot_general(cur2, w_ref[...], _DN,
                                 preferred_element_type=_F32)


def _kb_res(cur, r0, r1, b, w):
    ko = w.shape[1]
    return pl.pallas_call(
        _kb_res_kernel,
        out_shape=(jax.ShapeDtypeStruct((M, ko), _F32),
                   jax.ShapeDtypeStruct((M, 32), _F32)),
    )(cur, r0, r1, b, w)


def _kf_kernel(e0_ref, e1_ref, b_ref, o_ref):
    o_ref[...] = e0_ref[...] + e1_ref[...] + b_ref[...]


def _kf(e0, e1, b):
    return pl.pallas_call(
        _kf_kernel,
        out_shape=jax.ShapeDtypeStruct((M, 16), _F32),
    )(e0, e1, b)


def _mesh():
    return plsc.VectorSubcoreMesh(core_axis_name="c", subcore_axis_name="s",
                                  num_cores=NC, num_subcores=NS)


def _sc_edge_conv(table, src, kid, dst, zeros, *, K, acc_rows, C,
                  split_cores, idx_core_stride, super_=8):
    Ep = src.shape[0]
    n_workers = NC * NS if split_cores else NS
    per_w = Ep // n_workers
    assert per_w % CHUNK == 0 and acc_rows % (NS * 8) == 0
    SUPER = super_
    n_super = per_w // (SUPER * CHUNK)
    n_tail = (per_w - n_super * SUPER * CHUNK) // CHUNK
    rps = acc_rows // NS
    jpc = CHUNK // 16

    @functools.partial(
        pl.kernel,
        out_type=jax.ShapeDtypeStruct((NC * acc_rows, C), _F32),
        mesh=_mesh(),
        scratch_types=[
            pltpu.VMEM((SUPER * CHUNK,), jnp.int32),
            pltpu.VMEM((SUPER * CHUNK,), jnp.int32),
            pltpu.VMEM((SUPER * CHUNK,), jnp.int32),
            pltpu.VMEM((SUPER, CHUNK), jnp.int32),
            pltpu.VMEM((SUPER, CHUNK), jnp.int32),
            pltpu.VMEM((SUPER, CHUNK, C), _F32),
            pltpu.VMEM_SHARED((acc_rows, C), _F32),
            pltpu.SemaphoreType.DMA,
            pltpu.SemaphoreType.DMA,
        ],
        compiler_params=pltpu.CompilerParams(use_tc_tiling_on_sc=False),
    )
    def k(table_h, src_h, kid_h, dst_h, zeros_h, out_h,
          src_v, kid_v, dst_v, idx2, dst2, rows_v, acc, gsem, ssem):
        c = lax.axis_index("c")
        s = lax.axis_index("s")
        pltpu.sync_copy(zeros_h.at[pl.ds(0, rps)],
                        acc.at[pl.ds(s * rps, rps)])
        plsc.subcore_barrier()
        if split_cores:
            base = (s * NC + c) * per_w
        else:
            base = s * per_w

        def run(e0, nsl):
            n = nsl * CHUNK
            pltpu.sync_copy(src_h.at[pl.ds(e0, n)], src_v.at[pl.ds(0, n)])
            pltpu.sync_copy(kid_h.at[pl.ds(e0, n)], kid_v.at[pl.ds(0, n)])
            pltpu.sync_copy(dst_h.at[pl.ds(e0, n)], dst_v.at[pl.ds(0, n)])
            for j in range(nsl * jpc):
                sl = pl.ds(j * 16, 16)
                idx = src_v[sl] * K + kid_v[sl]
                if idx_core_stride:
                    idx = idx + c * idx_core_stride
                idx2[j // jpc, pl.ds((j % jpc) * 16, 16)] = idx
                dst2[j // jpc, pl.ds((j % jpc) * 16, 16)] = dst_v[sl]
            gds = [pltpu.async_copy(table_h.at[idx2.at[u]], rows_v.at[u],
                                    gsem) for u in range(nsl)]
            sds = []
            for u in range(nsl):
                gds[u].wait()
                sds.append(pltpu.async_copy(rows_v.at[u],
                                            acc.at[dst2.at[u]], ssem,
                                            add=True))
            for sd in sds:
                sd.wait()

        def body(t, carry):
            run(base + t * (SUPER * CHUNK), SUPER)
            return carry

        lax.fori_loop(0, n_super, body, 0)
        if n_tail:
            run(base + n_super * SUPER * CHUNK, n_tail)
        plsc.subcore_barrier()
        pltpu.sync_copy(acc.at[pl.ds(s * rps, rps)],
                        out_h.at[pl.ds(c * acc_rows + s * rps, rps)])

    return k(table, src, kid, dst, zeros)


def _pad_edges(src, kid, dst, n_workers, dummy_row):
    per = n_workers * CHUNK
    E = src.shape[0]
    Ep = ((E + per - 1) // per) * per
    pad = Ep - E
    src = jnp.concatenate([src, jnp.zeros((pad,), jnp.int32)])
    kid = jnp.concatenate([kid, jnp.zeros((pad,), jnp.int32)])
    dst = jnp.concatenate([dst, jnp.full((pad,), dummy_row, jnp.int32)])
    return src, kid, dst


def kernel(x, edge_index1, kid1, src_d, dst_d, kid_d, edge_index3, kid3,
           W1, b1, Wd, bd, Wra, bra, Wrb, brb, W4, b4):
    BL = Wra.shape[0]
    NPAD = 50048
    MPAD = 6272

    w1_stacked = jnp.stack([
        W1[:, :, :32].transpose(1, 0, 2).reshape(64, 27 * 32),
        W1[:, :, 32:].transpose(1, 0, 2).reshape(64, 27 * 32)])
    wd2d = Wd.transpose(1, 0, 2).reshape(64, K2 * 32)
    wra2d = Wra.transpose(0, 2, 1, 3).reshape(BL, 32, 27 * 32)
    wrb2d = Wrb.transpose(0, 2, 1, 3).reshape(BL, 32, 27 * 32)
    w4p = jnp.pad(W4.transpose(1, 0, 2), ((0, 0), (0, 0), (0, 8)))
    w4p = w4p.reshape(32, 27 * 16)
    b1a = b1[:32].reshape(1, 32)
    b1b = b1[32:].reshape(1, 32)
    bd2 = bd.reshape(1, 32)
    b4p = jnp.pad(b4, (0, 8)).reshape(1, 16)
    zeros32 = jnp.zeros((NPAD // NS, 32), _F32)
    zeros16 = jnp.zeros((MPAD // NS, 16), _F32)

    s1, k1, d1 = _pad_edges(edge_index1[0], kid1, edge_index1[1], NS, N)
    sd, kd, dd = _pad_edges(src_d, kid_d, dst_d, NC * NS, M)
    s3, k3, d3 = _pad_edges(edge_index3[0], kid3, edge_index3[1], NC * NS, M)

    h1 = _conv1_tables(x, w1_stacked, 2000)
    table1 = h1.reshape(NC * N * K3, 32)
    p1 = _sc_edge_conv(table1, s1, k1, d1, zeros32, K=K3, acc_rows=NPAD,
                       C=32, split_cores=False, idx_core_stride=N * K3,
                       super_=4)
    p1 = p1.reshape(NC, NPAD, 32)[:, :N]

    hd = _kb_down(p1[0], p1[1], b1a, b1b, wd2d[:32], wd2d[32:], 2000)
    qd = _sc_edge_conv(hd.reshape(N * K2, 32), sd, kd, dd, zeros32[:MPAD // NS],
                       K=K2, acc_rows=MPAD, C=32, split_cores=True,
                       idx_core_stride=0)
    qd = qd.reshape(NC, MPAD, 32)[:, :M]

    ha, cur = _kb_first(qd[0], qd[1], bd2, wra2d[0])
    for i in range(BL):
        ra = _sc_edge_conv(ha.reshape(M * K3, 32), s3, k3, d3,
                           zeros32[:MPAD // NS], K=K3, acc_rows=MPAD, C=32,
                           split_cores=True, idx_core_stride=0)
        ra = ra.reshape(NC, MPAD, 32)[:, :M]
        hb = _kb_mid(ra[0], ra[1], bra[i].reshape(1, 32), wrb2d[i])
        rb = _sc_edge_conv(hb.reshape(M * K3, 32), s3, k3, d3,
                           zeros32[:MPAD // NS], K=K3, acc_rows=MPAD, C=32,
                           split_cores=True, idx_core_stride=0)
        rb = rb.reshape(NC, MPAD, 32)[:, :M]
        wnext = wra2d[i + 1] if i + 1 < BL else w4p
        ha, cur = _kb_res(cur, rb[0], rb[1], brb[i].reshape(1, 32), wnext)

    e = _sc_edge_conv(ha.reshape(M * K3, 16), s3, k3, d3, zeros16,
                      K=K3, acc_rows=MPAD, C=16, split_cores=True,
                      idx_core_stride=0)
    e = e.reshape(NC, MPAD, 16)[:, :M]
    out = _kf(e[0], e[1], b4p)
    return out[:, :8]
